# baseline (speedup 1.0000x reference)
import functools

import jax
import jax.numpy as jnp
from jax import lax
from jax.experimental import pallas as pl
from jax.experimental.pallas import tpu as pltpu
from jax.experimental.pallas import tpu_sc as plsc

VOCAB_ROWS = 1000001
EMBED_DIM = 64
BATCH = 16384

_SPLIT = 491520

_CHUNK_C = 32768
_TC_GRID = -(-(VOCAB_ROWS - _SPLIT) // _CHUNK_C)
_QUARTER = _CHUNK_C // 4
_TC_ROWS = _TC_GRID * _QUARTER
_TC_OFF = _SPLIT // _CHUNK_C

_NC = 2
_NS = 16
_NW = _NC * _NS
_SC_COLS = _SPLIT // 128
_COLS_PER_W = _SC_COLS // _NW
_SC_ROWS = _SC_COLS * 32

_B_PER_W = BATCH // _NW
_IDX_CHUNK = 128
_N_CHUNKS = _B_PER_W // _IDX_CHUNK


def _bf16_lo(v):
    return (v + jnp.uint32(0x7FFF) + ((v >> 16) & jnp.uint32(1))) >> 16


def _bf16_hi(v):
    return (v + jnp.uint32(0x7FFF) + ((v >> 16) & jnp.uint32(1))) & jnp.uint32(
        0xFFFF0000
    )


def _pack_block(tT_ref, out_ref):
    x = tT_ref[...]
    xt = jnp.swapaxes(x, 0, 1)
    u = lax.bitcast_convert_type(xt, jnp.uint32)
    a = u[0 * _QUARTER : 1 * _QUARTER]
    b = u[1 * _QUARTER : 2 * _QUARTER]
    c = u[2 * _QUARTER : 3 * _QUARTER]
    d = u[3 * _QUARTER : 4 * _QUARTER]
    p01 = _bf16_lo(a) | _bf16_hi(b)
    p23 = _bf16_lo(c) | _bf16_hi(d)
    out_ref[:, :EMBED_DIM] = lax.bitcast_convert_type(p01, jnp.float32)
    out_ref[:, EMBED_DIM:] = lax.bitcast_convert_type(p23, jnp.float32)


def _tc_pack(tableT):
    return pl.pallas_call(
        _pack_block,
        grid=(_TC_GRID,),
        in_specs=[
            pl.BlockSpec((EMBED_DIM, _CHUNK_C), lambda i: (0, i + _TC_OFF))
        ],
        out_specs=pl.BlockSpec((_QUARTER, 128), lambda i: (i, 0)),
        out_shape=jax.ShapeDtypeStruct((_TC_ROWS, 128), jnp.float32),
    )(tableT)


def _sc_pack(table3):
    mesh = plsc.VectorSubcoreMesh(core_axis_name="c", subcore_axis_name="s")

    @functools.partial(
        pl.kernel,
        mesh=mesh,
        compiler_params=pltpu.CompilerParams(
            use_tc_tiling_on_sc=True, needs_layout_passes=False
        ),
        out_type=jax.ShapeDtypeStruct((_SC_ROWS, 128), jnp.float32),
        scratch_types=[
            pltpu.VMEM((8, 8, 128), jnp.float32),
            pltpu.VMEM((8, 8, 128), jnp.float32),
            pltpu.VMEM((32, 128), jnp.float32),
            pltpu.VMEM((32, 128), jnp.float32),
            pltpu.SemaphoreType.DMA,
            pltpu.SemaphoreType.DMA,
        ],
    )
    def pack_kernel(t3_hbm, out_hbm, tb0, tb1, ob0, ob1, sem0, sem1):
        wid = lax.axis_index("s") * _NC + lax.axis_index("c")
        base_col = wid * _COLS_PER_W

        def fire(col, tb, sem):
            coff = pl.multiple_of(col * 128, 128)
            for g in range(8):
                pltpu.async_copy(
                    t3_hbm.at[g, :, pl.ds(coff, 128)], tb.at[g], sem
                )

        def drain(tb, sem):
            for g in range(8):
                pltpu.make_async_copy(
                    t3_hbm.at[g, :, pl.ds(0, 128)], tb.at[g], sem
                ).wait()

        def compute(col, tb, ob):
            iota = lax.iota(jnp.int32, 16)
            for k in range(32):
                for jj in range(4):
                    gi = (iota + 16 * jj) // 8
                    ri = (iota + 16 * jj) % 8
                    v0 = plsc.load_gather(tb, [gi, ri, jnp.full((16,), k, jnp.int32)])
                    v1 = plsc.load_gather(tb, [gi, ri, jnp.full((16,), k + 32, jnp.int32)])
                    v2 = plsc.load_gather(tb, [gi, ri, jnp.full((16,), k + 64, jnp.int32)])
                    v3 = plsc.load_gather(tb, [gi, ri, jnp.full((16,), k + 96, jnp.int32)])
                    u0 = lax.bitcast_convert_type(v0, jnp.uint32)
                    u1 = lax.bitcast_convert_type(v1, jnp.uint32)
                    u2 = lax.bitcast_convert_type(v2, jnp.uint32)
                    u3b = lax.bitcast_convert_type(v3, jnp.uint32)
                    p01 = _bf16_lo(u0) | _bf16_hi(u1)
                    p23 = _bf16_lo(u2) | _bf16_hi(u3b)
                    ob[k, pl.ds(16 * jj, 16)] = lax.bitcast_convert_type(
                        p01, jnp.float32
                    )
                    ob[k, pl.ds(64 + 16 * jj, 16)] = lax.bitcast_convert_type(
                        p23, jnp.float32
                    )

        def writeback(col, ob):
            roff = pl.multiple_of(col * 32, 32)
            pltpu.sync_copy(ob, out_hbm.at[pl.ds(roff, 32)])

        fire(base_col, tb0, sem0)
        fire(base_col + 1, tb1, sem1)

        def body(i, carry):
            col0 = base_col + 2 * i
            drain(tb0, sem0)
            compute(col0, tb0, ob0)
            writeback(col0, ob0)
            fire(col0 + 2, tb0, sem0)
            drain(tb1, sem1)
            compute(col0 + 1, tb1, ob1)
            writeback(col0 + 1, ob1)
            fire(col0 + 3, tb1, sem1)
            return carry

        lax.fori_loop(0, _COLS_PER_W // 2, body, 0)
        drain(tb0, sem0)
        drain(tb1, sem1)

    return pack_kernel(table3)


def _sc_gather(tc_packed, sc_packed, slot_a3d, slot_b3d):
    mesh = plsc.VectorSubcoreMesh(core_axis_name="c", subcore_axis_name="s")

    @functools.partial(
        pl.kernel,
        mesh=mesh,
        compiler_params=pltpu.CompilerParams(use_tc_tiling_on_sc=True),
        out_type=(
            jax.ShapeDtypeStruct((BATCH, 128), jnp.float32),
            jax.ShapeDtypeStruct((BATCH, 128), jnp.float32),
        ),
        scratch_types=[
            pltpu.VMEM((_N_CHUNKS, _IDX_CHUNK), jnp.int32),
            pltpu.VMEM((_N_CHUNKS, _IDX_CHUNK), jnp.int32),
            pltpu.VMEM((_B_PER_W, 128), jnp.float32),
            pltpu.SemaphoreType.DMA,
        ],
    )
    def gather_kernel(tcp_hbm, scp_hbm, ia_hbm, ib_hbm, outa_hbm, outb_hbm,
                      ia_v, ib_v, rows_v, sem):
        wid = lax.axis_index("s") * _NC + lax.axis_index("c")
        base = wid * _B_PER_W
        pltpu.sync_copy(ia_hbm.at[wid], ia_v)
        pltpu.sync_copy(ib_hbm.at[wid], ib_v)
        copies = []
        for j in range(_N_CHUNKS):
            copies.append(
                pltpu.async_copy(
                    tcp_hbm.at[ia_v.at[j]],
                    rows_v.at[pl.ds(j * _IDX_CHUNK, _IDX_CHUNK)],
                    sem,
                )
            )
        for c in copies:
            c.wait()
        pltpu.sync_copy(rows_v, outa_hbm.at[pl.ds(base, _B_PER_W)])
        copies = []
        for j in range(_N_CHUNKS):
            copies.append(
                pltpu.async_copy(
                    scp_hbm.at[ib_v.at[j]],
                    rows_v.at[pl.ds(j * _IDX_CHUNK, _IDX_CHUNK)],
                    sem,
                )
            )
        for c in copies:
            c.wait()
        pltpu.sync_copy(rows_v, outb_hbm.at[pl.ds(base, _B_PER_W)])

    return gather_kernel(tc_packed, sc_packed, slot_a3d, slot_b3d)


def _mlp_block(xa_ref, xb_ref, src_ref, sel_ref, shf_ref, w1_ref, b1_ref,
               w2_ref, b2_ref, w3_ref, b3_ref, o_ref):
    xa = lax.bitcast_convert_type(xa_ref[...], jnp.uint32)
    xb = lax.bitcast_convert_type(xb_ref[...], jnp.uint32)
    x = jnp.where(src_ref[...] != 0, xb, xa)
    half = jnp.where(sel_ref[...] != 0, x[:, EMBED_DIM:], x[:, :EMBED_DIM])
    bits = (half >> shf_ref[...].astype(jnp.uint32)) & jnp.uint32(0xFFFF)
    emb = lax.bitcast_convert_type(
        bits.astype(jnp.uint16), jnp.bfloat16
    ).astype(jnp.float32)
    h = jnp.maximum(
        jnp.dot(emb, w1_ref[...], preferred_element_type=jnp.float32)
        + b1_ref[...],
        0.0,
    )
    h = jnp.maximum(
        jnp.dot(h, w2_ref[...], preferred_element_type=jnp.float32)
        + b2_ref[...],
        0.0,
    )
    o_ref[...] = (
        jnp.dot(h, w3_ref[...], preferred_element_type=jnp.float32)
        + b3_ref[...]
    )


def _tc_mlp(xa, xb, src, sel, shf, W1, b1, W2, b2, W3, b3):
    blk = 2048
    grid = (BATCH // blk,)
    return pl.pallas_call(
        _mlp_block,
        grid=grid,
        in_specs=[
            pl.BlockSpec((blk, 128), lambda i: (i, 0)),
            pl.BlockSpec((blk, 128), lambda i: (i, 0)),
            pl.BlockSpec((blk, 1), lambda i: (i, 0)),
            pl.BlockSpec((blk, 1), lambda i: (i, 0)),
            pl.BlockSpec((blk, 1), lambda i: (i, 0)),
            pl.BlockSpec(W1.shape, lambda i: (0, 0)),
            pl.BlockSpec(b1.shape, lambda i: (0, 0)),
            pl.BlockSpec(W2.shape, lambda i: (0, 0)),
            pl.BlockSpec(b2.shape, lambda i: (0, 0)),
            pl.BlockSpec(W3.shape, lambda i: (0, 0)),
            pl.BlockSpec(b3.shape, lambda i: (0, 0)),
        ],
        out_specs=pl.BlockSpec((blk, W3.shape[1]), lambda i: (i, 0)),
        out_shape=jax.ShapeDtypeStruct((BATCH, W3.shape[1]), jnp.float32),
    )(xa, xb, src, sel, shf, W1, b1, W2, b2, W3, b3)


def kernel(user_id, table, W1, b1, W2, b2, W3, b3):
    uid = user_id.astype(jnp.int32)
    in_sc = uid < _SPLIT
    u2 = jnp.maximum(uid - _SPLIT, 0)
    r = u2 % _CHUNK_C
    sub_tc = r // _QUARTER
    slot_tc = (u2 // _CHUNK_C) * _QUARTER + r % _QUARTER
    sub_sc = (uid % 128) // 32
    slot_sc = (uid // 128) * 32 + uid % 32
    slot_a3d = jnp.where(in_sc, 0, slot_tc).reshape(_NW, _N_CHUNKS, _IDX_CHUNK)
    slot_b3d = jnp.where(in_sc, slot_sc, 0).reshape(_NW, _N_CHUNKS, _IDX_CHUNK)
    sub = jnp.where(in_sc, sub_sc, sub_tc)
    src = in_sc.astype(jnp.int32).reshape(BATCH, 1)
    sel = (sub >> 1).reshape(BATCH, 1)
    shf = ((sub & 1) * 16).reshape(BATCH, 1)

    tableT = table.T
    table3 = tableT.reshape(8, 8, VOCAB_ROWS)
    tc_packed = _tc_pack(tableT)
    sc_packed = _sc_pack(table3)
    rows_a, rows_b = _sc_gather(tc_packed, sc_packed, slot_a3d, slot_b3d)
    return _tc_mlp(
        rows_a,
        rows_b,
        src,
        sel,
        shf,
        W1,
        b1.reshape(1, -1),
        W2,
        b2.reshape(1, -1),
        W3,
        b3.reshape(1, -1),
    )


# speedup vs baseline: 5.8924x; 5.8924x over previous
import functools

import jax
import jax.numpy as jnp
from jax import lax
from jax.experimental import pallas as pl
from jax.experimental.pallas import tpu as pltpu
from jax.experimental.pallas import tpu_sc as plsc

VOCAB_ROWS = 1000001
EMBED_DIM = 64
BATCH = 16384

_CHUNK_C = 65536
_GRID_A = -(-VOCAB_ROWS // _CHUNK_C)
_QUARTER = _CHUNK_C // 4
_PACK_ROWS = _GRID_A * _QUARTER

_NC = 2
_NS = 16
_NW = _NC * _NS
_B_PER_W = BATCH // _NW
_IDX_CHUNK = 128
_N_CHUNKS = _B_PER_W // _IDX_CHUNK


def _bf16_lo(v):
    return (v + jnp.uint32(0x7FFF) + ((v >> 16) & jnp.uint32(1))) >> 16


def _bf16_hi(v):
    return (v + jnp.uint32(0x7FFF) + ((v >> 16) & jnp.uint32(1))) & jnp.uint32(
        0xFFFF0000
    )


def _pack_block(tT_ref, out_ref):
    x = tT_ref[...]
    xt = jnp.swapaxes(x, 0, 1)
    u = lax.bitcast_convert_type(xt, jnp.uint32)
    a = u[0 * _QUARTER : 1 * _QUARTER]
    b = u[1 * _QUARTER : 2 * _QUARTER]
    c = u[2 * _QUARTER : 3 * _QUARTER]
    d = u[3 * _QUARTER : 4 * _QUARTER]
    p01 = _bf16_lo(a) | _bf16_hi(b)
    p23 = _bf16_lo(c) | _bf16_hi(d)
    out_ref[:, :EMBED_DIM] = lax.bitcast_convert_type(p01, jnp.float32)
    out_ref[:, EMBED_DIM:] = lax.bitcast_convert_type(p23, jnp.float32)


def _tc_pack(tableT):
    return pl.pallas_call(
        _pack_block,
        grid=(_GRID_A,),
        in_specs=[pl.BlockSpec((EMBED_DIM, _CHUNK_C), lambda i: (0, i))],
        out_specs=pl.BlockSpec((_QUARTER, 128), lambda i: (i, 0)),
        out_shape=jax.ShapeDtypeStruct((_PACK_ROWS, 128), jnp.float32),
    )(tableT)


def _sc_gather(packed, slot3d):
    mesh = plsc.VectorSubcoreMesh(core_axis_name="c", subcore_axis_name="s")

    @functools.partial(
        pl.kernel,
        mesh=mesh,
        compiler_params=pltpu.CompilerParams(use_tc_tiling_on_sc=True),
        out_type=jax.ShapeDtypeStruct((BATCH, 128), jnp.float32),
        scratch_types=[
            pltpu.VMEM((_N_CHUNKS, _IDX_CHUNK), jnp.int32),
            pltpu.VMEM((_B_PER_W, 128), jnp.float32),
            pltpu.SemaphoreType.DMA,
        ],
    )
    def gather_kernel(packed_hbm, idx_hbm, out_hbm, idx_v, rows_v, sem):
        wid = lax.axis_index("s") * _NC + lax.axis_index("c")
        base = wid * _B_PER_W
        pltpu.sync_copy(idx_hbm.at[wid], idx_v)
        copies = []
        for j in range(_N_CHUNKS):
            copies.append(
                pltpu.async_copy(
                    packed_hbm.at[idx_v.at[j]],
                    rows_v.at[pl.ds(j * _IDX_CHUNK, _IDX_CHUNK)],
                    sem,
                )
            )
        for c in copies:
            c.wait()
        pltpu.sync_copy(rows_v, out_hbm.at[pl.ds(base, _B_PER_W)])

    return gather_kernel(packed, slot3d)


def _mlp_block(x_ref, sel_ref, shf_ref, w1_ref, b1_ref, w2_ref, b2_ref,
               w3_ref, b3_ref, o_ref):
    x = lax.bitcast_convert_type(x_ref[...], jnp.uint32)
    half = jnp.where(sel_ref[...] != 0, x[:, EMBED_DIM:], x[:, :EMBED_DIM])
    bits = (half >> shf_ref[...].astype(jnp.uint32)) & jnp.uint32(0xFFFF)
    emb = lax.bitcast_convert_type(
        bits.astype(jnp.uint16), jnp.bfloat16
    ).astype(jnp.float32)
    h = jnp.maximum(
        jnp.dot(emb, w1_ref[...], preferred_element_type=jnp.float32)
        + b1_ref[...],
        0.0,
    )
    h = jnp.maximum(
        jnp.dot(h, w2_ref[...], preferred_element_type=jnp.float32)
        + b2_ref[...],
        0.0,
    )
    o_ref[...] = (
        jnp.dot(h, w3_ref[...], preferred_element_type=jnp.float32)
        + b3_ref[...]
    )


def _tc_mlp(x, sel, shf, W1, b1, W2, b2, W3, b3):
    blk = 4096
    grid = (BATCH // blk,)
    return pl.pallas_call(
        _mlp_block,
        grid=grid,
        in_specs=[
            pl.BlockSpec((blk, 128), lambda i: (i, 0)),
            pl.BlockSpec((blk, 1), lambda i: (i, 0)),
            pl.BlockSpec((blk, 1), lambda i: (i, 0)),
            pl.BlockSpec(W1.shape, lambda i: (0, 0)),
            pl.BlockSpec(b1.shape, lambda i: (0, 0)),
            pl.BlockSpec(W2.shape, lambda i: (0, 0)),
            pl.BlockSpec(b2.shape, lambda i: (0, 0)),
            pl.BlockSpec(W3.shape, lambda i: (0, 0)),
            pl.BlockSpec(b3.shape, lambda i: (0, 0)),
        ],
        out_specs=pl.BlockSpec((blk, W3.shape[1]), lambda i: (i, 0)),
        out_shape=jax.ShapeDtypeStruct((BATCH, W3.shape[1]), jnp.float32),
    )(x, sel, shf, W1, b1, W2, b2, W3, b3)


def kernel(user_id, table, W1, b1, W2, b2, W3, b3):
    uid = user_id.astype(jnp.int32)
    chunk = uid // _CHUNK_C
    r = uid % _CHUNK_C
    sub = r // _QUARTER
    slot3d = (chunk * _QUARTER + r % _QUARTER).reshape(
        _NW, _N_CHUNKS, _IDX_CHUNK
    )
    sel = (sub >> 1).reshape(BATCH, 1)
    shf = ((sub & 1) * 16).reshape(BATCH, 1)
    packed = _tc_pack(table.T)
    rows = _sc_gather(packed, slot3d)
    return _tc_mlp(
        rows,
        sel,
        shf,
        W1,
        b1.reshape(1, -1),
        W2,
        b2.reshape(1, -1),
        W3,
        b3.reshape(1, -1),
    )


# speedup vs baseline: 6.0932x; 1.0341x over previous
import functools

import jax
import jax.numpy as jnp
from jax import lax
from jax.experimental import pallas as pl
from jax.experimental.pallas import tpu as pltpu
from jax.experimental.pallas import tpu_sc as plsc

VOCAB_ROWS = 1000001
EMBED_DIM = 64
BATCH = 16384

_CHUNK_C = 32768
_GRID_A = -(-VOCAB_ROWS // _CHUNK_C)
_QUARTER = _CHUNK_C // 4
_PACK_ROWS = _GRID_A * _QUARTER

_NC = 2
_NS = 16
_NW = _NC * _NS
_B_PER_W = BATCH // _NW
_IDX_CHUNK = 128
_N_CHUNKS = _B_PER_W // _IDX_CHUNK


def _bf16_lo(v):
    return (v + jnp.uint32(0x7FFF) + ((v >> 16) & jnp.uint32(1))) >> 16


def _bf16_hi(v):
    return (v + jnp.uint32(0x7FFF) + ((v >> 16) & jnp.uint32(1))) & jnp.uint32(
        0xFFFF0000
    )


def _pack_block(tT_ref, out_ref):
    x = tT_ref[...]
    xt = jnp.swapaxes(x, 0, 1)
    u = lax.bitcast_convert_type(xt, jnp.uint32)
    a = u[0 * _QUARTER : 1 * _QUARTER]
    b = u[1 * _QUARTER : 2 * _QUARTER]
    c = u[2 * _QUARTER : 3 * _QUARTER]
    d = u[3 * _QUARTER : 4 * _QUARTER]
    p01 = _bf16_lo(a) | _bf16_hi(b)
    p23 = _bf16_lo(c) | _bf16_hi(d)
    out_ref[:, :EMBED_DIM] = lax.bitcast_convert_type(p01, jnp.float32)
    out_ref[:, EMBED_DIM:] = lax.bitcast_convert_type(p23, jnp.float32)


def _tc_pack(tableT):
    return pl.pallas_call(
        _pack_block,
        grid=(_GRID_A,),
        in_specs=[pl.BlockSpec((EMBED_DIM, _CHUNK_C), lambda i: (0, i))],
        out_specs=pl.BlockSpec((_QUARTER, 128), lambda i: (i, 0)),
        out_shape=jax.ShapeDtypeStruct((_PACK_ROWS, 128), jnp.float32),
    )(tableT)


def _sc_gather(packed, slot3d):
    mesh = plsc.VectorSubcoreMesh(core_axis_name="c", subcore_axis_name="s")

    @functools.partial(
        pl.kernel,
        mesh=mesh,
        compiler_params=pltpu.CompilerParams(use_tc_tiling_on_sc=True),
        out_type=jax.ShapeDtypeStruct((BATCH, 128), jnp.float32),
        scratch_types=[
            pltpu.VMEM((_N_CHUNKS, _IDX_CHUNK), jnp.int32),
            pltpu.VMEM((_B_PER_W, 128), jnp.float32),
            pltpu.SemaphoreType.DMA,
        ],
    )
    def gather_kernel(packed_hbm, idx_hbm, out_hbm, idx_v, rows_v, sem):
        wid = lax.axis_index("s") * _NC + lax.axis_index("c")
        base = wid * _B_PER_W
        pltpu.sync_copy(idx_hbm.at[wid], idx_v)
        copies = []
        for j in range(_N_CHUNKS):
            copies.append(
                pltpu.async_copy(
                    packed_hbm.at[idx_v.at[j]],
                    rows_v.at[pl.ds(j * _IDX_CHUNK, _IDX_CHUNK)],
                    sem,
                )
            )
        for c in copies:
            c.wait()
        pltpu.sync_copy(rows_v, out_hbm.at[pl.ds(base, _B_PER_W)])

    return gather_kernel(packed, slot3d)


def _mlp_block(x_ref, sel_ref, shf_ref, w1_ref, b1_ref, w2_ref, b2_ref,
               w3_ref, b3_ref, o_ref):
    x = lax.bitcast_convert_type(x_ref[...], jnp.uint32)
    half = jnp.where(sel_ref[...] != 0, x[:, EMBED_DIM:], x[:, :EMBED_DIM])
    bits = (half >> shf_ref[...].astype(jnp.uint32)) & jnp.uint32(0xFFFF)
    emb = lax.bitcast_convert_type(
        bits.astype(jnp.uint16), jnp.bfloat16
    ).astype(jnp.float32)
    h = jnp.maximum(
        jnp.dot(emb, w1_ref[...], preferred_element_type=jnp.float32)
        + b1_ref[...],
        0.0,
    )
    h = jnp.maximum(
        jnp.dot(h, w2_ref[...], preferred_element_type=jnp.float32)
        + b2_ref[...],
        0.0,
    )
    o_ref[...] = (
        jnp.dot(h, w3_ref[...], preferred_element_type=jnp.float32)
        + b3_ref[...]
    )


def _tc_mlp(x, sel, shf, W1, b1, W2, b2, W3, b3):
    blk = 4096
    grid = (BATCH // blk,)
    return pl.pallas_call(
        _mlp_block,
        grid=grid,
        in_specs=[
            pl.BlockSpec((blk, 128), lambda i: (i, 0)),
            pl.BlockSpec((blk, 1), lambda i: (i, 0)),
            pl.BlockSpec((blk, 1), lambda i: (i, 0)),
            pl.BlockSpec(W1.shape, lambda i: (0, 0)),
            pl.BlockSpec(b1.shape, lambda i: (0, 0)),
            pl.BlockSpec(W2.shape, lambda i: (0, 0)),
            pl.BlockSpec(b2.shape, lambda i: (0, 0)),
            pl.BlockSpec(W3.shape, lambda i: (0, 0)),
            pl.BlockSpec(b3.shape, lambda i: (0, 0)),
        ],
        out_specs=pl.BlockSpec((blk, W3.shape[1]), lambda i: (i, 0)),
        out_shape=jax.ShapeDtypeStruct((BATCH, W3.shape[1]), jnp.float32),
    )(x, sel, shf, W1, b1, W2, b2, W3, b3)


def kernel(user_id, table, W1, b1, W2, b2, W3, b3):
    uid = user_id.astype(jnp.int32)
    chunk = uid // _CHUNK_C
    r = uid % _CHUNK_C
    sub = r // _QUARTER
    slot3d = (chunk * _QUARTER + r % _QUARTER).reshape(
        _NW, _N_CHUNKS, _IDX_CHUNK
    )
    sel = (sub >> 1).reshape(BATCH, 1)
    shf = ((sub & 1) * 16).reshape(BATCH, 1)
    packed = _tc_pack(table.T)
    rows = _sc_gather(packed, slot3d)
    return _tc_mlp(
        rows,
        sel,
        shf,
        W1,
        b1.reshape(1, -1),
        W2,
        b2.reshape(1, -1),
        W3,
        b3.reshape(1, -1),
    )


# speedup vs baseline: 6.3022x; 1.0343x over previous
import functools

import jax
import jax.numpy as jnp
from jax import lax
from jax.experimental import pallas as pl
from jax.experimental.pallas import tpu as pltpu
from jax.experimental.pallas import tpu_sc as plsc

VOCAB_ROWS = 1000001
EMBED_DIM = 64
BATCH = 16384

_CHUNK_C = 32768
_GRID_A = -(-VOCAB_ROWS // _CHUNK_C)
_QUARTER = _CHUNK_C // 4
_PACK_ROWS = _GRID_A * _QUARTER

_NC = 2
_NS = 16
_NW = _NC * _NS
_B_PER_W = BATCH // _NW
_IDX_CHUNK = 128
_N_CHUNKS = _B_PER_W // _IDX_CHUNK


def _bf16_lo(v):
    return (v + jnp.uint32(0x7FFF) + ((v >> 16) & jnp.uint32(1))) >> 16


def _bf16_hi(v):
    return (v + jnp.uint32(0x7FFF) + ((v >> 16) & jnp.uint32(1))) & jnp.uint32(
        0xFFFF0000
    )


def _pack_block(tT_ref, out_ref):
    x = tT_ref[...]
    xt = jnp.swapaxes(x, 0, 1)
    u = lax.bitcast_convert_type(xt, jnp.uint32)
    a = u[0 * _QUARTER : 1 * _QUARTER]
    b = u[1 * _QUARTER : 2 * _QUARTER]
    c = u[2 * _QUARTER : 3 * _QUARTER]
    d = u[3 * _QUARTER : 4 * _QUARTER]
    p01 = _bf16_lo(a) | _bf16_hi(b)
    p23 = _bf16_lo(c) | _bf16_hi(d)
    out_ref[:, :EMBED_DIM] = lax.bitcast_convert_type(p01, jnp.float32)
    out_ref[:, EMBED_DIM:] = lax.bitcast_convert_type(p23, jnp.float32)


def _tc_pack(tableT):
    return pl.pallas_call(
        _pack_block,
        grid=(_GRID_A,),
        in_specs=[pl.BlockSpec((EMBED_DIM, _CHUNK_C), lambda i: (0, i))],
        out_specs=pl.BlockSpec((_QUARTER, 128), lambda i: (i, 0)),
        out_shape=jax.ShapeDtypeStruct((_PACK_ROWS, 128), jnp.float32),
    )(tableT)


def _sc_gather(packed, slot3d):
    mesh = plsc.VectorSubcoreMesh(core_axis_name="c", subcore_axis_name="s")

    @functools.partial(
        pl.kernel,
        mesh=mesh,
        compiler_params=pltpu.CompilerParams(use_tc_tiling_on_sc=True),
        out_type=jax.ShapeDtypeStruct((BATCH, 128), jnp.float32),
        scratch_types=[
            pltpu.VMEM((_N_CHUNKS, _IDX_CHUNK), jnp.int32),
            pltpu.VMEM((_B_PER_W, 128), jnp.float32),
            pltpu.SemaphoreType.DMA,
        ],
    )
    def gather_kernel(packed_hbm, idx_hbm, out_hbm, idx_v, rows_v, sem):
        wid = lax.axis_index("s") * _NC + lax.axis_index("c")
        base = wid * _B_PER_W
        pltpu.sync_copy(idx_hbm.at[wid], idx_v)
        copies = []
        for j in range(_N_CHUNKS):
            copies.append(
                pltpu.async_copy(
                    packed_hbm.at[idx_v.at[j]],
                    rows_v.at[pl.ds(j * _IDX_CHUNK, _IDX_CHUNK)],
                    sem,
                )
            )
        for c in copies:
            c.wait()
        pltpu.sync_copy(rows_v, out_hbm.at[pl.ds(base, _B_PER_W)])

    return gather_kernel(packed, slot3d)


def _mlp_block(x_ref, uid_ref, w1_ref, b1_ref, w2_ref, b2_ref,
               w3_ref, b3_ref, o_ref):
    x = lax.bitcast_convert_type(x_ref[...], jnp.uint32)
    sub = (uid_ref[...] % _CHUNK_C) // _QUARTER
    half = jnp.where(sub >= 2, x[:, EMBED_DIM:], x[:, :EMBED_DIM])
    shift = ((sub & 1) * 16).astype(jnp.uint32)
    bits = (half >> shift) & jnp.uint32(0xFFFF)
    emb = lax.bitcast_convert_type(
        bits.astype(jnp.uint16), jnp.bfloat16
    ).astype(jnp.float32)
    embT = jnp.swapaxes(emb, 0, 1)
    dn = (((0,), (0,)), ((), ()))
    h = jnp.maximum(
        lax.dot_general(w1_ref[...], embT, dn,
                        preferred_element_type=jnp.float32)
        + b1_ref[...],
        0.0,
    )
    h = jnp.maximum(
        lax.dot_general(w2_ref[...], h, dn,
                        preferred_element_type=jnp.float32)
        + b2_ref[...],
        0.0,
    )
    o_ref[...] = (
        lax.dot_general(w3_ref[...], h, dn,
                        preferred_element_type=jnp.float32)
        + b3_ref[...]
    )


def _tc_mlp(x, uid2d, W1, b1, W2, b2, W3, b3):
    blk = 4096
    grid = (BATCH // blk,)
    return pl.pallas_call(
        _mlp_block,
        grid=grid,
        in_specs=[
            pl.BlockSpec((blk, 128), lambda i: (i, 0)),
            pl.BlockSpec((blk, 1), lambda i: (i, 0)),
            pl.BlockSpec(W1.shape, lambda i: (0, 0)),
            pl.BlockSpec(b1.shape, lambda i: (0, 0)),
            pl.BlockSpec(W2.shape, lambda i: (0, 0)),
            pl.BlockSpec(b2.shape, lambda i: (0, 0)),
            pl.BlockSpec(W3.shape, lambda i: (0, 0)),
            pl.BlockSpec(b3.shape, lambda i: (0, 0)),
        ],
        out_specs=pl.BlockSpec((W3.shape[1], blk), lambda i: (0, i)),
        out_shape=jax.ShapeDtypeStruct((W3.shape[1], BATCH), jnp.float32),
    )(x, uid2d, W1, b1, W2, b2, W3, b3)


def kernel(user_id, table, W1, b1, W2, b2, W3, b3):
    uid = user_id.astype(jnp.int32)
    chunk = uid // _CHUNK_C
    r = uid % _CHUNK_C
    slot3d = (chunk * _QUARTER + r % _QUARTER).reshape(
        _NW, _N_CHUNKS, _IDX_CHUNK
    )
    packed = _tc_pack(table.T)
    rows = _sc_gather(packed, slot3d)
    outT = _tc_mlp(
        rows,
        uid.reshape(BATCH, 1),
        W1,
        b1.reshape(-1, 1),
        W2,
        b2.reshape(-1, 1),
        W3,
        b3.reshape(-1, 1),
    )
    return outT.T
